# baseline (speedup 1.0000x reference)
import functools

import jax
import jax.numpy as jnp
import numpy as np
from jax import lax
from jax.experimental import pallas as pl
from jax.experimental.pallas import tpu as pltpu
from jax.experimental.pallas import tpu_sc as plsc

_B = 16384
_F = 26
_H = 16
_NC = 2
_NS = 16
_NW = _NC * _NS
_CHUNK = 128
_NCHUNKS = _B // _CHUNK
_CPW = _NCHUNKS // _NW
_G = _CHUNK // 16


def _fm_body(xot_hbm, fc_hbm, embt_hbm, bias_hbm, dummy_hbm, out_hbm,
             idx_v, dest_v, lin_v, bias_v, out_v, sem_emb, sem_lin):
    c = lax.axis_index("c")
    s = lax.axis_index("s")
    wid = s * _NC + c

    pltpu.sync_copy(bias_hbm, bias_v)
    bias_vec = bias_v[...]

    def do_chunk(ci, carry):
        chunk = wid * _CPW + ci
        pltpu.sync_copy(xot_hbm.at[:, pl.ds(chunk * _CHUNK, _CHUNK)], idx_v)

        lin_handles = [
            pltpu.async_copy(fc_hbm.at[idx_v.at[f]], lin_v.at[f], sem_lin)
            for f in range(_F)
        ]

        def fire(h, c3):
            plane = embt_hbm.at[h]
            for f in range(_F):
                pltpu.async_copy(plane.at[idx_v.at[f]],
                                 dest_v.at[f].at[h], sem_emb)
            return c3

        lax.fori_loop(0, _H, fire, 0)
        pltpu.make_async_copy(dummy_hbm, dest_v, sem_emb).wait()
        for h in lin_handles:
            h.wait()

        def group(g, c4):
            fmacc = jnp.zeros((16,), jnp.float32)
            for h in range(_H):
                a = jnp.zeros((16,), jnp.float32)
                q = jnp.zeros((16,), jnp.float32)
                for f in range(_F):
                    v = dest_v[f, h, pl.ds(g * 16, 16)]
                    a = a + v
                    q = q + v * v
                fmacc = fmacc + (a * a - q)
            lin_acc = bias_vec
            for f in range(_F):
                lin_acc = lin_acc + lin_v[f, pl.ds(g * 16, 16)]
            z = 0.5 * fmacc + lin_acc
            out_v[pl.ds(g * 16, 16)] = 1.0 / (1.0 + jnp.exp(-z))
            return c4

        lax.fori_loop(0, _G, group, 0)
        pltpu.sync_copy(out_v, out_hbm.at[pl.ds(chunk * _CHUNK, _CHUNK)])
        return carry

    lax.fori_loop(0, _CPW, do_chunk, 0)


@functools.cache
def _build_fm_kernel():
    return pl.kernel(
        _fm_body,
        mesh=plsc.VectorSubcoreMesh(core_axis_name="c", subcore_axis_name="s"),
        compiler_params=pltpu.CompilerParams(
            needs_layout_passes=False, use_tc_tiling_on_sc=False),
        out_type=jax.ShapeDtypeStruct((_B,), jnp.float32),
        scratch_types=[
            pltpu.VMEM((_F, _CHUNK), jnp.int32),
            pltpu.VMEM((_F, _H, _CHUNK), jnp.float32),
            pltpu.VMEM((_F, _CHUNK), jnp.float32),
            pltpu.VMEM((16,), jnp.float32),
            pltpu.VMEM((_CHUNK,), jnp.float32),
            pltpu.SemaphoreType.DMA,
            pltpu.SemaphoreType.DMA,
        ],
    )


def kernel(x, fc_w, embed_w, bias):
    offs = np.arange(_F, dtype=np.int32) * 100000
    xo_t = (x.astype(jnp.int32) + jnp.asarray(offs)[None, :]).T
    emb_t = embed_w.T
    fc_flat = fc_w.reshape(-1)
    bias16 = jnp.broadcast_to(bias, (16,)).astype(jnp.float32)
    dummy = jnp.zeros((_F, _H, _CHUNK), jnp.float32)
    return _build_fm_kernel()(xo_t, fc_flat, emb_t, bias16, dummy)


# speedup vs baseline: 2.1471x; 2.1471x over previous
import functools

import jax
import jax.numpy as jnp
import numpy as np
from jax import lax
from jax.experimental import pallas as pl
from jax.experimental.pallas import tpu as pltpu
from jax.experimental.pallas import tpu_sc as plsc

_B = 16384
_F = 26
_H = 16
_E = 2600000
_NC = 2
_NS = 16
_NW = _NC * _NS
_CHUNK = 128
_NCHUNKS = _B // _CHUNK
_CPW = _NCHUNKS // _NW
_G = _CHUNK // 16

_RC = 4096
_RBLK = -(-_E // _RC)


def _relayout_body(in_ref, out_ref):
    out_ref[...] = in_ref[...].T


@functools.cache
def _build_relayout():
    return pl.pallas_call(
        _relayout_body,
        grid=(_RBLK,),
        in_specs=[pl.BlockSpec((_H, _RC), lambda j: (0, j))],
        out_specs=pl.BlockSpec((_RC, _H), lambda j: (j, 0)),
        out_shape=jax.ShapeDtypeStruct((_E, _H), jnp.float32),
    )


def _fm_body(xot_hbm, fc_hbm, emb_hbm, bias_hbm, out_hbm,
             idx0, idx1, rows0, rows1, lin0, lin1, bias_v, out_v, tbuf,
             sem0, sem1):
    c = lax.axis_index("c")
    s = lax.axis_index("s")
    wid = s * _NC + c

    pltpu.sync_copy(bias_hbm, bias_v)
    bias_vec = bias_v[...]
    lane = lax.iota(jnp.int32, 16)

    idx_bufs = (idx0, idx1)
    rows_bufs = (rows0, rows1)
    lin_bufs = (lin0, lin1)
    sems = (sem0, sem1)

    def fire(ci, k):
        chunk = wid * _CPW + ci
        pltpu.sync_copy(xot_hbm.at[:, pl.ds(chunk * _CHUNK, _CHUNK)],
                        idx_bufs[k])
        handles = []
        for f in range(_F):
            handles.append(pltpu.async_copy(
                emb_hbm.at[idx_bufs[k].at[f]], rows_bufs[k].at[f], sems[k]))
            handles.append(pltpu.async_copy(
                fc_hbm.at[idx_bufs[k].at[f]], lin_bufs[k].at[f], sems[k]))
        return handles

    def compute(ci, k):
        rows_v = rows_bufs[k]
        lin_v = lin_bufs[k]

        def group(g, carry):
            lin_acc = bias_vec
            for f in range(_F):
                lin_acc = lin_acc + lin_v[f, pl.ds(g * 16, 16)]

            def sample(l, c4):
                j = g * 16 + l
                a = jnp.zeros((16,), jnp.float32)
                q = jnp.zeros((16,), jnp.float32)
                for f in range(_F):
                    v = rows_v[f, j, :]
                    a = a + v
                    q = q + v * v
                tbuf[l, :] = a * a - q
                return c4

            lax.fori_loop(0, 16, sample, 0)
            acc = jnp.zeros((16,), jnp.float32)
            for h in range(16):
                col = plsc.load_gather(tbuf, [lane, jnp.full((16,), h, jnp.int32)])
                acc = acc + col
            z = 0.5 * acc + lin_acc
            out_v[pl.ds(g * 16, 16)] = 1.0 / (1.0 + jnp.exp(-z))
            return carry

        lax.fori_loop(0, _G, group, 0)
        chunk = wid * _CPW + ci
        pltpu.sync_copy(out_v, out_hbm.at[pl.ds(chunk * _CHUNK, _CHUNK)])

    handles = fire(0, 0)
    for ci in range(_CPW):
        nxt = fire(ci + 1, (ci + 1) % 2) if ci + 1 < _CPW else None
        for h in handles:
            h.wait()
        compute(ci, ci % 2)
        handles = nxt


@functools.cache
def _build_fm_kernel():
    return pl.kernel(
        _fm_body,
        mesh=plsc.VectorSubcoreMesh(core_axis_name="c", subcore_axis_name="s"),
        compiler_params=pltpu.CompilerParams(
            needs_layout_passes=False, use_tc_tiling_on_sc=False),
        out_type=jax.ShapeDtypeStruct((_B,), jnp.float32),
        scratch_types=[
            pltpu.VMEM((_F, _CHUNK), jnp.int32),
            pltpu.VMEM((_F, _CHUNK), jnp.int32),
            pltpu.VMEM((_F, _CHUNK, _H), jnp.float32),
            pltpu.VMEM((_F, _CHUNK, _H), jnp.float32),
            pltpu.VMEM((_F, _CHUNK), jnp.float32),
            pltpu.VMEM((_F, _CHUNK), jnp.float32),
            pltpu.VMEM((16,), jnp.float32),
            pltpu.VMEM((_CHUNK,), jnp.float32),
            pltpu.VMEM((16, 16), jnp.float32),
            pltpu.SemaphoreType.DMA,
            pltpu.SemaphoreType.DMA,
        ],
    )


def kernel(x, fc_w, embed_w, bias):
    offs = np.arange(_F, dtype=np.int32) * 100000
    xo_t = (x.astype(jnp.int32) + jnp.asarray(offs)[None, :]).T
    emb_t = embed_w.T
    fc_flat = fc_w.reshape(-1)
    bias16 = jnp.broadcast_to(bias, (16,)).astype(jnp.float32)
    emb_rm = _build_relayout()(emb_t)
    return _build_fm_kernel()(xo_t, fc_flat, emb_rm, bias16)
